# baseline (speedup 1.0000x reference)
import functools

import jax
import jax.numpy as jnp
from jax import lax
from jax.experimental import pallas as pl
from jax.experimental.pallas import tpu as pltpu
from jax.experimental.pallas import tpu_sc as plsc

B = 1024
L = 50
NC = 20
K = L * NC
KP = 1024
D = 64
DP = 128
NCHUNK = 8
CHUNK = KP // NCHUNK
NQ = 4
QITEMS = KP // NQ
CPQ = NCHUNK // NQ
BB = 128
NCORES = 2
NSUB = 16
NW = NCORES * NSUB
RPW = B // NW


def _prep_body(dm_ref, mm_ref, lm_ref, dt_ref, mt_ref, lt_ref, demo_ref,
               wdt_ref, bdt_ref, wmt_ref, bmt_ref, wlt_ref, blt_ref,
               wdemo_ref, bdemo_ref,
               wd_ref, wm_ref, wl_ref, tvd_ref, tvm_ref, tvl_ref, dv_ref):
    rid = lax.broadcasted_iota(jnp.int32, (KP, L), 0)
    gid = lax.broadcasted_iota(jnp.int32, (KP, L), 1)
    G = ((rid // NC) == gid).astype(jnp.float32)

    def channel(mask, t, W, bvec, w_ref, tv_ref):
        lensum = lax.dot_general(mask, G, (((1,), (0,)), ((), ())),
                                 preferred_element_type=jnp.float32)
        lm = (lensum > 0.0).astype(jnp.float32)
        lm_exp = lax.dot_general(lm, G, (((1,), (1,)), ((), ())),
                                 preferred_element_type=jnp.float32)
        w_ref[...] = mask * lm_exp * (1.0 / K)
        s1 = jnp.sum(t * lm, axis=1, keepdims=True) * (1.0 / L)
        s2 = jnp.sum(lm, axis=1, keepdims=True) * (1.0 / L)
        tv_ref[...] = s1 * W + s2 * bvec

    channel(dm_ref[...], dt_ref[...], wdt_ref[...], bdt_ref[...], wd_ref, tvd_ref)
    channel(mm_ref[...], mt_ref[...], wmt_ref[...], bmt_ref[...], wm_ref, tvm_ref)
    channel(lm_ref[...], lt_ref[...], wlt_ref[...], blt_ref[...], wl_ref, tvl_ref)
    dv_ref[...] = jnp.dot(demo_ref[...], wdemo_ref[...],
                          preferred_element_type=jnp.float32) + bdemo_ref[...]


def _prep(masks, times, demo, wdt, bdt, wmt, bmt, wlt, blt, wdemo, bdemo):
    nblk = B // BB
    row_spec = lambda cols: pl.BlockSpec((BB, cols), lambda i: (i, 0))
    full_spec = lambda r, c: pl.BlockSpec((r, c), lambda i: (0, 0))
    out_shapes = (
        [jax.ShapeDtypeStruct((B, KP), jnp.float32)] * 3
        + [jax.ShapeDtypeStruct((B, 16), jnp.float32)] * 3
        + [jax.ShapeDtypeStruct((B, D), jnp.float32)]
    )
    in_specs = (
        [row_spec(KP)] * 3 + [row_spec(L)] * 3 + [row_spec(4)]
        + [full_spec(1, 16), full_spec(1, 16)] * 3
        + [full_spec(4, D), full_spec(1, D)]
    )
    out_specs = [row_spec(KP)] * 3 + [row_spec(16)] * 3 + [row_spec(D)]
    return pl.pallas_call(
        _prep_body,
        grid=(nblk,),
        in_specs=in_specs,
        out_specs=out_specs,
        out_shape=out_shapes,
    )(*masks, *times, demo, wdt, bdt, wmt, bmt, wlt, blt, wdemo, bdemo)


_GDN = lax.GatherDimensionNumbers(offset_dims=(), collapsed_slice_dims=(0,),
                                  start_index_map=(0,))


def _bcast(vec, j):
    idx = jnp.full((16, 1), j, jnp.int32)
    return lax.gather(vec, idx, _GDN, slice_sizes=(1,),
                      mode=lax.GatherScatterMode.PROMISE_IN_BOUNDS)


NBUF = 3
LABV = 1000 * D


def _pool_body(dtab, mtab, ltab, didx, midx, lidx, wdh, wmh, wlh,
               out_d, out_m, out_l, idx_v, w_v, rows_v, ltab_v, out_v, *sems):
    wid = lax.axis_index("s") * NCORES + lax.axis_index("c")
    base = wid * RPW
    zero = jnp.zeros((16,), jnp.float32)

    for p in range(8):
        pltpu.sync_copy(ltab.at[pl.ds(p * (LABV // 8), LABV // 8)],
                        ltab_v.at[pl.ds(p * (LABV // 8), LABV // 8)])
    offs = [lax.iota(jnp.int32, 16) + q * 16 for q in range(4)]

    def lab_row(i, carry):
        r = base + i
        pltpu.sync_copy(lidx.at[r], idx_v)
        pltpu.sync_copy(wlh.at[r], w_v)

        def mac(g, a):
            ivec = idx_v[g // 8, pl.ds((g % 8) * 16, 16)]
            wvec = w_v[pl.ds(g * 16, 16)]
            for j in range(16):
                wb = _bcast(wvec, j)
                sb = ivec[j] * D
                a = tuple(a[q] + wb * ltab_v[pl.ds(sb + q * 16, 16)]
                          for q in range(4))
            return a

        acc = lax.fori_loop(0, KP // 16, mac, (zero, zero, zero, zero))
        for q in range(4):
            out_v[i, pl.ds(q * 16, 16)] = acc[q]
        return carry

    lax.fori_loop(0, RPW, lab_row, 0)
    pltpu.sync_copy(out_v, out_l.at[pl.ds(base, RPW)])

    for tab, idx_hbm, w_hbm, out_hbm in ((dtab, didx, wdh, out_d),
                                         (mtab, midx, wmh, out_m)):
        def row_step(i, carry, tab=tab, idx_hbm=idx_hbm, w_hbm=w_hbm):
            r = base + i
            pltpu.sync_copy(idx_hbm.at[r], idx_v)
            pltpu.sync_copy(w_hbm.at[r], w_v)

            def enqueue(c):
                s = c % NBUF
                return pltpu.async_copy(tab.at[idx_v.at[c]],
                                        rows_v.at[s], sems[s])

            handles = [enqueue(c) for c in range(NBUF)]
            acc = (zero, zero, zero, zero)
            for c in range(NCHUNK):
                handles[c % NBUF].wait()
                buf = rows_v.at[c % NBUF]

                def mac(g, a, c=c, buf=buf):
                    wvec = w_v[pl.ds(c * CHUNK + g * 16, 16)]
                    rowbase = g * 16
                    for j in range(16):
                        wb = _bcast(wvec, j)
                        a = tuple(a[q] + wb * buf[rowbase + j,
                                                  pl.ds(q * 16, 16)]
                                  for q in range(4))
                    return a

                acc = lax.fori_loop(0, CHUNK // 16, mac, acc)
                if c + NBUF < NCHUNK:
                    handles[c % NBUF] = enqueue(c + NBUF)
            for q in range(4):
                out_v[i, pl.ds(q * 16, 16)] = acc[q]
            return carry

        lax.fori_loop(0, RPW, row_step, 0)
        pltpu.sync_copy(out_v, out_hbm.at[pl.ds(base, RPW)])


def _pool(dtab, mtab, ltab, didx, midx, lidx, wd, wm, wl):
    mesh = plsc.VectorSubcoreMesh(core_axis_name="c", subcore_axis_name="s",
                                  num_cores=NCORES, num_subcores=NSUB)
    kern = pl.kernel(
        _pool_body,
        out_type=[jax.ShapeDtypeStruct((B, D), jnp.float32)] * 3,
        mesh=mesh,
        scratch_types=[
            pltpu.VMEM((NCHUNK, CHUNK), jnp.int32),
            pltpu.VMEM((KP,), jnp.float32),
            pltpu.VMEM((NBUF, CHUNK, DP), jnp.float32),
            pltpu.VMEM((LABV,), jnp.float32),
            pltpu.VMEM((RPW, D), jnp.float32),
        ] + [pltpu.SemaphoreType.DMA] * NBUF,
    )
    return kern(dtab, mtab, ltab, didx, midx, lidx, wd, wm, wl)


def _mlp_body(x_ref, w_ref, b_ref, o_ref):
    o_ref[...] = jnp.dot(jnp.maximum(x_ref[...], 0.0), w_ref[...],
                         preferred_element_type=jnp.float32) + b_ref[...]


def _mlp(x, W_h, b_h):
    nblk = B // BB
    h_in, h_out = W_h.shape
    return pl.pallas_call(
        _mlp_body,
        grid=(nblk,),
        in_specs=[pl.BlockSpec((BB, h_in), lambda i: (i, 0)),
                  pl.BlockSpec((h_in, h_out), lambda i: (0, 0)),
                  pl.BlockSpec((1, h_out), lambda i: (0, 0))],
        out_specs=pl.BlockSpec((BB, h_out), lambda i: (i, 0)),
        out_shape=jax.ShapeDtypeStruct((B, h_out), jnp.float32),
    )(x, W_h, b_h)


def _flat_pad(x):
    flat = x.reshape(B, K)
    return jnp.pad(flat, ((0, 0), (0, KP - K)))


def kernel(diag_seq, diag_time, diag_mask, med_seq, med_time, med_mask,
           lab_seq, lab_time, lab_mask, demo,
           diag_table, med_table, lab_table,
           W_dt, b_dt, W_mt, b_mt, W_lt, b_lt,
           W_demo, b_demo, W_h, b_h):
    masks = [_flat_pad(m) for m in (diag_mask, med_mask, lab_mask)]
    idxs = [_flat_pad(s).reshape(B, NCHUNK, CHUNK)
            for s in (diag_seq, med_seq, lab_seq)]
    tabs = [jnp.pad(t, ((0, 0), (0, DP - D)))
            for t in (diag_table, med_table)] + [lab_table.reshape(-1)]

    wd, wm, wl, tvd, tvm, tvl, dv = _prep(
        masks, [diag_time, med_time, lab_time], demo,
        W_dt, b_dt.reshape(1, 16), W_mt, b_mt.reshape(1, 16),
        W_lt, b_lt.reshape(1, 16), W_demo, b_demo.reshape(1, D))

    emb_d, emb_m, emb_l = _pool(*tabs, *idxs, wd, wm, wl)

    rep = jnp.concatenate([emb_d, tvd, emb_m, tvm, emb_l, tvl, dv], axis=-1)
    return _mlp(rep, W_h, b_h.reshape(1, -1))


# speedup vs baseline: 1.0432x; 1.0432x over previous
import functools

import jax
import jax.numpy as jnp
from jax import lax
from jax.experimental import pallas as pl
from jax.experimental.pallas import tpu as pltpu
from jax.experimental.pallas import tpu_sc as plsc

B = 1024
L = 50
NC = 20
K = L * NC
KP = 1024
D = 64
DP = 128
NCHUNK = 8
CHUNK = KP // NCHUNK
NQ = 4
QITEMS = KP // NQ
CPQ = NCHUNK // NQ
BB = 128
NCORES = 2
NSUB = 16
NW = NCORES * NSUB
RPW = B // NW


def _prep_body(dm_ref, mm_ref, lm_ref, dt_ref, mt_ref, lt_ref, demo_ref,
               wdt_ref, bdt_ref, wmt_ref, bmt_ref, wlt_ref, blt_ref,
               wdemo_ref, bdemo_ref,
               wd_ref, wm_ref, wl_ref, tvd_ref, tvm_ref, tvl_ref, dv_ref):
    rid = lax.broadcasted_iota(jnp.int32, (KP, L), 0)
    gid = lax.broadcasted_iota(jnp.int32, (KP, L), 1)
    G = ((rid // NC) == gid).astype(jnp.float32)

    def channel(mask, t, W, bvec, w_ref, tv_ref):
        lensum = lax.dot_general(mask, G, (((1,), (0,)), ((), ())),
                                 preferred_element_type=jnp.float32)
        lm = (lensum > 0.0).astype(jnp.float32)
        lm_exp = lax.dot_general(lm, G, (((1,), (1,)), ((), ())),
                                 preferred_element_type=jnp.float32)
        w_ref[...] = mask * lm_exp * (1.0 / K)
        s1 = jnp.sum(t * lm, axis=1, keepdims=True) * (1.0 / L)
        s2 = jnp.sum(lm, axis=1, keepdims=True) * (1.0 / L)
        tv_ref[...] = s1 * W + s2 * bvec

    channel(dm_ref[...], dt_ref[...], wdt_ref[...], bdt_ref[...], wd_ref, tvd_ref)
    channel(mm_ref[...], mt_ref[...], wmt_ref[...], bmt_ref[...], wm_ref, tvm_ref)
    channel(lm_ref[...], lt_ref[...], wlt_ref[...], blt_ref[...], wl_ref, tvl_ref)
    dv_ref[...] = jnp.dot(demo_ref[...], wdemo_ref[...],
                          preferred_element_type=jnp.float32) + bdemo_ref[...]


def _prep(masks, times, demo, wdt, bdt, wmt, bmt, wlt, blt, wdemo, bdemo):
    nblk = B // BB
    row_spec = lambda cols: pl.BlockSpec((BB, cols), lambda i: (i, 0))
    full_spec = lambda r, c: pl.BlockSpec((r, c), lambda i: (0, 0))
    out_shapes = (
        [jax.ShapeDtypeStruct((B, KP), jnp.float32)] * 3
        + [jax.ShapeDtypeStruct((B, 16), jnp.float32)] * 3
        + [jax.ShapeDtypeStruct((B, D), jnp.float32)]
    )
    in_specs = (
        [row_spec(KP)] * 3 + [row_spec(L)] * 3 + [row_spec(4)]
        + [full_spec(1, 16), full_spec(1, 16)] * 3
        + [full_spec(4, D), full_spec(1, D)]
    )
    out_specs = [row_spec(KP)] * 3 + [row_spec(16)] * 3 + [row_spec(D)]
    return pl.pallas_call(
        _prep_body,
        grid=(nblk,),
        in_specs=in_specs,
        out_specs=out_specs,
        out_shape=out_shapes,
    )(*masks, *times, demo, wdt, bdt, wmt, bmt, wlt, blt, wdemo, bdemo)


_GDN = lax.GatherDimensionNumbers(offset_dims=(), collapsed_slice_dims=(0,),
                                  start_index_map=(0,))


def _bcast(vec, j):
    idx = jnp.full((16, 1), j, jnp.int32)
    return lax.gather(vec, idx, _GDN, slice_sizes=(1,),
                      mode=lax.GatherScatterMode.PROMISE_IN_BOUNDS)


NBUF = 3
LABV = 1000 * D


def _pool_body(dtab, mtab, ltab, didx, midx, lidx, wdh, wmh, wlh,
               out_d, out_m, out_l, idx_v, w_v, idx_l, w_l,
               rows_v, ltab_v, out_v, out_lv, *sems):
    wid = lax.axis_index("s") * NCORES + lax.axis_index("c")
    base = wid * RPW
    last = base + RPW - 1
    zero = jnp.zeros((16,), jnp.float32)
    sem_i, sem_w = sems[NBUF], sems[NBUF + 1]

    for p in range(8):
        pltpu.sync_copy(ltab.at[pl.ds(p * (LABV // 8), LABV // 8)],
                        ltab_v.at[pl.ds(p * (LABV // 8), LABV // 8)])

    def lab_mac(i):
        r = base + i
        pltpu.sync_copy(lidx.at[r], idx_l)
        pltpu.sync_copy(wlh.at[r], w_l)

        def mac(g, a):
            ivec = idx_l[g // 8, pl.ds((g % 8) * 16, 16)]
            wvec = w_l[pl.ds(g * 16, 16)]
            for j in range(16):
                wb = _bcast(wvec, j)
                sb = ivec[j] * D
                a = tuple(a[q] + wb * ltab_v[pl.ds(sb + q * 16, 16)]
                          for q in range(4))
            return a

        acc = lax.fori_loop(0, KP // 16, mac, (zero, zero, zero, zero))
        for q in range(4):
            out_lv[i, pl.ds(q * 16, 16)] = acc[q]

    for tab, idx_hbm, w_hbm, out_hbm, with_lab in (
            (dtab, didx, wdh, out_d, True), (mtab, midx, wmh, out_m, False)):
        pltpu.async_copy(idx_hbm.at[base], idx_v.at[0], sem_i)
        pltpu.async_copy(w_hbm.at[base], w_v.at[0], sem_w)

        def row_step(i, carry, tab=tab, idx_hbm=idx_hbm, w_hbm=w_hbm,
                     with_lab=with_lab):
            r = base + i
            s = lax.rem(i, 2)
            sn = 1 - s
            rn = jnp.minimum(r + 1, last)
            pltpu.make_async_copy(idx_hbm.at[r], idx_v.at[s], sem_i).wait()
            pltpu.make_async_copy(w_hbm.at[r], w_v.at[s], sem_w).wait()
            idx_row = idx_v.at[s]
            w_row = w_v.at[s]

            def enqueue(c):
                b = c % NBUF
                return pltpu.async_copy(tab.at[idx_row.at[c]],
                                        rows_v.at[b], sems[b])

            handles = [enqueue(c) for c in range(NBUF)]
            pltpu.async_copy(idx_hbm.at[rn], idx_v.at[sn], sem_i)
            pltpu.async_copy(w_hbm.at[rn], w_v.at[sn], sem_w)
            if with_lab:
                lab_mac(i)
            acc = (zero, zero, zero, zero)
            for c in range(NCHUNK):
                handles[c % NBUF].wait()
                buf = rows_v.at[c % NBUF]

                def mac(g, a, c=c, buf=buf):
                    wvec = w_row[pl.ds(c * CHUNK + g * 16, 16)]
                    rowbase = g * 16
                    for j in range(16):
                        wb = _bcast(wvec, j)
                        a = tuple(a[q] + wb * buf[rowbase + j,
                                                  pl.ds(q * 16, 16)]
                                  for q in range(4))
                    return a

                acc = lax.fori_loop(0, CHUNK // 16, mac, acc)
                if c + NBUF < NCHUNK:
                    handles[c % NBUF] = enqueue(c + NBUF)
            for q in range(4):
                out_v[i, pl.ds(q * 16, 16)] = acc[q]
            return carry

        lax.fori_loop(0, RPW, row_step, 0)
        pltpu.make_async_copy(idx_hbm.at[last], idx_v.at[RPW % 2], sem_i).wait()
        pltpu.make_async_copy(w_hbm.at[last], w_v.at[RPW % 2], sem_w).wait()
        pltpu.sync_copy(out_v, out_hbm.at[pl.ds(base, RPW)])
    pltpu.sync_copy(out_lv, out_l.at[pl.ds(base, RPW)])


def _pool(dtab, mtab, ltab, didx, midx, lidx, wd, wm, wl):
    mesh = plsc.VectorSubcoreMesh(core_axis_name="c", subcore_axis_name="s",
                                  num_cores=NCORES, num_subcores=NSUB)
    kern = pl.kernel(
        _pool_body,
        out_type=[jax.ShapeDtypeStruct((B, D), jnp.float32)] * 3,
        mesh=mesh,
        scratch_types=[
            pltpu.VMEM((2, NCHUNK, CHUNK), jnp.int32),
            pltpu.VMEM((2, KP), jnp.float32),
            pltpu.VMEM((NCHUNK, CHUNK), jnp.int32),
            pltpu.VMEM((KP,), jnp.float32),
            pltpu.VMEM((NBUF, CHUNK, DP), jnp.float32),
            pltpu.VMEM((LABV,), jnp.float32),
            pltpu.VMEM((RPW, D), jnp.float32),
            pltpu.VMEM((RPW, D), jnp.float32),
        ] + [pltpu.SemaphoreType.DMA] * (NBUF + 2),
    )
    return kern(dtab, mtab, ltab, didx, midx, lidx, wd, wm, wl)


def _mlp_body(x_ref, w_ref, b_ref, o_ref):
    o_ref[...] = jnp.dot(jnp.maximum(x_ref[...], 0.0), w_ref[...],
                         preferred_element_type=jnp.float32) + b_ref[...]


def _mlp(x, W_h, b_h):
    nblk = B // BB
    h_in, h_out = W_h.shape
    return pl.pallas_call(
        _mlp_body,
        grid=(nblk,),
        in_specs=[pl.BlockSpec((BB, h_in), lambda i: (i, 0)),
                  pl.BlockSpec((h_in, h_out), lambda i: (0, 0)),
                  pl.BlockSpec((1, h_out), lambda i: (0, 0))],
        out_specs=pl.BlockSpec((BB, h_out), lambda i: (i, 0)),
        out_shape=jax.ShapeDtypeStruct((B, h_out), jnp.float32),
    )(x, W_h, b_h)


def _flat_pad(x):
    flat = x.reshape(B, K)
    return jnp.pad(flat, ((0, 0), (0, KP - K)))


def kernel(diag_seq, diag_time, diag_mask, med_seq, med_time, med_mask,
           lab_seq, lab_time, lab_mask, demo,
           diag_table, med_table, lab_table,
           W_dt, b_dt, W_mt, b_mt, W_lt, b_lt,
           W_demo, b_demo, W_h, b_h):
    masks = [_flat_pad(m) for m in (diag_mask, med_mask, lab_mask)]
    idxs = [_flat_pad(s).reshape(B, NCHUNK, CHUNK)
            for s in (diag_seq, med_seq, lab_seq)]
    tabs = [jnp.pad(t, ((0, 0), (0, DP - D)))
            for t in (diag_table, med_table)] + [lab_table.reshape(-1)]

    wd, wm, wl, tvd, tvm, tvl, dv = _prep(
        masks, [diag_time, med_time, lab_time], demo,
        W_dt, b_dt.reshape(1, 16), W_mt, b_mt.reshape(1, 16),
        W_lt, b_lt.reshape(1, 16), W_demo, b_demo.reshape(1, D))

    emb_d, emb_m, emb_l = _pool(*tabs, *idxs, wd, wm, wl)

    rep = jnp.concatenate([emb_d, tvd, emb_m, tvm, emb_l, tvl, dv], axis=-1)
    return _mlp(rep, W_h, b_h.reshape(1, -1))
